# Initial kernel scaffold; baseline (speedup 1.0000x reference)
import jax
import jax.numpy as jnp
from jax.experimental import pallas as pl


def kernel(uv_inputs, data):
    raise NotImplementedError("write your pallas kernel here")


# speedup vs baseline: 21.5559x; 21.5559x over previous
import functools

import jax
import jax.numpy as jnp
from jax import lax
from jax.experimental import pallas as pl
from jax.experimental.pallas import tpu as pltpu
from jax.experimental.pallas import tpu_sc as plsc

TEX = 1024
CH = 16
RES = 512
NPTS = RES * RES

NC, NS, L = 2, 16, 16
NW = NC * NS
BPW = NPTS // NW
CHUNK = 128
NCHUNK = BPW // CHUNK
NGRP = CHUNK // L

LEVELS = (
    (0, 1024),
    (1024 * 1024, 512),
    (1024 * 1024 + 512 * 512, 256),
    (1024 * 1024 + 512 * 512 + 256 * 256, 128),
)
NTAP = 16


def _floorf(v):
    f = v.astype(jnp.int32).astype(jnp.float32)
    return jnp.where(f > v, f - 1.0, f)


_mesh = plsc.VectorSubcoreMesh(
    core_axis_name="c", subcore_axis_name="s", num_cores=NC, num_subcores=NS
)


@functools.partial(
    pl.kernel,
    compiler_params=pltpu.CompilerParams(
        use_tc_tiling_on_sc=False, needs_layout_passes=False
    ),
    out_type=jax.ShapeDtypeStruct((CH, NPTS), jnp.float32),
    mesh=_mesh,
    scratch_types=[
        pltpu.VMEM((BPW,), jnp.float32),
        pltpu.VMEM((BPW,), jnp.float32),
        pltpu.VMEM((NTAP, CHUNK), jnp.int32),
        pltpu.VMEM((NTAP, CHUNK), jnp.float32),
        pltpu.VMEM((NTAP, CHUNK, CH), jnp.float32),
        pltpu.VMEM((CH, CHUNK), jnp.float32),
        pltpu.SemaphoreType.DMA,
    ],
)
def _sc_lookup(uvx_hbm, uvy_hbm, table_hbm, out_hbm, x_v, y_v, idx_v, w_v,
               rows_v, out_v, sem):
    wid = lax.axis_index("s") * NC + lax.axis_index("c")
    base = wid * BPW

    pltpu.sync_copy(uvx_hbm.at[pl.ds(base, BPW)], x_v)
    pltpu.sync_copy(uvy_hbm.at[pl.ds(base, BPW)], y_v)

    iota = lax.iota(jnp.int32, L)

    def chunk_body(c, _):
        co = c * CHUNK

        def calc_body(g, _):
            sl = pl.ds(g * L, L)
            px = x_v[pl.ds(co + g * L, L)]
            py = y_v[pl.ds(co + g * L, L)]
            for l, (tbase, w) in enumerate(LEVELS):
                half = w * 0.5
                off = (w - 1) * 0.5
                ixf = px * half + off
                iyf = py * half + off
                fx0 = _floorf(ixf)
                fy0 = _floorf(iyf)
                wx1 = ixf - fx0
                wy1 = iyf - fy0
                wx0 = 1.0 - wx1
                wy0 = 1.0 - wy1
                ix0 = fx0.astype(jnp.int32)
                iy0 = fy0.astype(jnp.int32)
                ix0c = jnp.clip(ix0, 0, w - 1)
                ix1c = jnp.clip(ix0 + 1, 0, w - 1)
                iy0c = jnp.clip(iy0, 0, w - 1)
                iy1c = jnp.clip(iy0 + 1, 0, w - 1)
                r0 = iy0c * w + tbase
                r1 = iy1c * w + tbase
                t = 4 * l
                idx_v[t + 0, sl] = r0 + ix0c
                idx_v[t + 1, sl] = r0 + ix1c
                idx_v[t + 2, sl] = r1 + ix0c
                idx_v[t + 3, sl] = r1 + ix1c
                w_v[t + 0, sl] = wy0 * wx0
                w_v[t + 1, sl] = wy0 * wx1
                w_v[t + 2, sl] = wy1 * wx0
                w_v[t + 3, sl] = wy1 * wx1
            return 0

        lax.fori_loop(0, NGRP, calc_body, 0)

        cps = [
            pltpu.async_copy(table_hbm.at[idx_v.at[t]], rows_v.at[t], sem)
            for t in range(NTAP)
        ]
        for cp in cps:
            cp.wait()

        def comb_body(g, _):
            sl = pl.ds(g * L, L)
            pidx = iota + g * L
            ws = [w_v[t, sl] for t in range(NTAP)]

            def ch_body(ch, carry):
                cws = carry
                cidx = jnp.broadcast_to(ch, (L,)).astype(jnp.int32)
                acc = [None, None, None, None]
                for t in range(NTAP):
                    v = plsc.load_gather(
                        rows_v,
                        [jnp.full((L,), t, jnp.int32), pidx, cidx],
                    )
                    a = t % 4
                    acc[a] = v * cws[t] if acc[a] is None else acc[a] + v * cws[t]
                out_v[ch, sl] = (acc[0] + acc[1]) + (acc[2] + acc[3])
                return cws

            lax.fori_loop(0, CH, ch_body, tuple(ws))
            return 0

        lax.fori_loop(0, NGRP, comb_body, 0)

        for ch in range(CH):
            pltpu.sync_copy(out_v.at[ch], out_hbm.at[ch, pl.ds(base + co, CHUNK)])
        return 0

    lax.fori_loop(0, NCHUNK, chunk_body, 0)


def _build_table(data):
    d = data[0]
    parts = []
    for yoff, w in ((0, 1024), (1024, 512), (1536, 256), (1792, 128)):
        parts.append(
            jnp.transpose(d[:, yoff:yoff + w, :w], (1, 2, 0)).reshape(w * w, CH)
        )
    return jnp.concatenate(parts, axis=0)


def kernel(uv_inputs, data):
    table = _build_table(data)
    uvx = uv_inputs[0, 0].reshape(NPTS)
    uvy = uv_inputs[0, 1].reshape(NPTS)
    out = _sc_lookup(uvx, uvy, table)
    return out.reshape(1, CH, RES, RES)


# speedup vs baseline: 24.6183x; 1.1421x over previous
import functools

import jax
import jax.numpy as jnp
from jax import lax
from jax.experimental import pallas as pl
from jax.experimental.pallas import tpu as pltpu
from jax.experimental.pallas import tpu_sc as plsc

TEX = 1024
CH = 16
RES = 512
NPTS = RES * RES

NC, NS, L = 2, 16, 16
NW = NC * NS
BPW = NPTS // NW
CHUNK = 128
NCHUNK = BPW // CHUNK
NPAIR = NCHUNK // 2
NGRP = CHUNK // L

LEVELS = (
    (0, 1024),
    (1024 * 1024, 512),
    (1024 * 1024 + 512 * 512, 256),
    (1024 * 1024 + 512 * 512 + 256 * 256, 128),
)
NTAP = 16


def _floorf(v):
    f = v.astype(jnp.int32).astype(jnp.float32)
    return jnp.where(f > v, f - 1.0, f)


_mesh = plsc.VectorSubcoreMesh(
    core_axis_name="c", subcore_axis_name="s", num_cores=NC, num_subcores=NS
)


@functools.partial(
    pl.kernel,
    compiler_params=pltpu.CompilerParams(
        use_tc_tiling_on_sc=False, needs_layout_passes=False
    ),
    out_type=jax.ShapeDtypeStruct((CH, NPTS), jnp.float32),
    mesh=_mesh,
    scratch_types=[
        pltpu.VMEM((BPW,), jnp.float32),
        pltpu.VMEM((BPW,), jnp.float32),
        pltpu.VMEM((2, NTAP, CHUNK), jnp.int32),
        pltpu.VMEM((2, NTAP, CHUNK), jnp.float32),
        pltpu.VMEM((2, NTAP, CHUNK, CH), jnp.float32),
        pltpu.VMEM((2, CH, CHUNK), jnp.float32),
        pltpu.SemaphoreType.DMA,
        pltpu.SemaphoreType.DMA,
        pltpu.SemaphoreType.DMA,
        pltpu.SemaphoreType.DMA,
    ],
)
def _sc_lookup(uvx_hbm, uvy_hbm, table_hbm, out_hbm, x_v, y_v, idx_v, w_v,
               rows_v, out_v, gsemA, gsemB, fsemA, fsemB):
    wid = lax.axis_index("s") * NC + lax.axis_index("c")
    base = wid * BPW

    pltpu.sync_copy(uvx_hbm.at[pl.ds(base, BPW)], x_v)
    pltpu.sync_copy(uvy_hbm.at[pl.ds(base, BPW)], y_v)

    iota = lax.iota(jnp.int32, L)

    def calc(co, b):
        def calc_body(g, _):
            sl = pl.ds(g * L, L)
            px = x_v[pl.ds(co + g * L, L)]
            py = y_v[pl.ds(co + g * L, L)]
            for l, (tbase, w) in enumerate(LEVELS):
                half = w * 0.5
                off = (w - 1) * 0.5
                ixf = px * half + off
                iyf = py * half + off
                fx0 = _floorf(ixf)
                fy0 = _floorf(iyf)
                wx1 = ixf - fx0
                wy1 = iyf - fy0
                wx0 = 1.0 - wx1
                wy0 = 1.0 - wy1
                ix0 = fx0.astype(jnp.int32)
                iy0 = fy0.astype(jnp.int32)
                ix0c = jnp.clip(ix0, 0, w - 1)
                ix1c = jnp.clip(ix0 + 1, 0, w - 1)
                iy0c = jnp.clip(iy0, 0, w - 1)
                iy1c = jnp.clip(iy0 + 1, 0, w - 1)
                r0 = iy0c * w + tbase
                r1 = iy1c * w + tbase
                t = 4 * l
                idx_v[b, t + 0, sl] = r0 + ix0c
                idx_v[b, t + 1, sl] = r0 + ix1c
                idx_v[b, t + 2, sl] = r1 + ix0c
                idx_v[b, t + 3, sl] = r1 + ix1c
                w_v[b, t + 0, sl] = wy0 * wx0
                w_v[b, t + 1, sl] = wy0 * wx1
                w_v[b, t + 2, sl] = wy1 * wx0
                w_v[b, t + 3, sl] = wy1 * wx1
            return 0

        lax.fori_loop(0, NGRP, calc_body, 0)

    def fire(b, gsem):
        for t in range(NTAP):
            pltpu.async_copy(table_hbm.at[idx_v.at[b, t]], rows_v.at[b, t], gsem)

    def drain_gathers(b, gsem):
        for t in range(NTAP):
            pltpu.make_async_copy(
                table_hbm.at[idx_v.at[b, t]], rows_v.at[b, t], gsem
            ).wait()

    def combine(b):
        def comb_body(g, _):
            sl = pl.ds(g * L, L)
            pidx = iota + g * L
            ws = [w_v[b, t, sl] for t in range(NTAP)]
            bs = jnp.full((L,), b, jnp.int32)
            for ch in range(CH):
                chs = jnp.full((L,), ch, jnp.int32)
                acc = [None, None, None, None]
                for t in range(NTAP):
                    v = plsc.load_gather(
                        rows_v, [bs, jnp.full((L,), t, jnp.int32), pidx, chs]
                    )
                    a = t % 4
                    acc[a] = v * ws[t] if acc[a] is None else acc[a] + v * ws[t]
                out_v[b, ch, sl] = (acc[0] + acc[1]) + (acc[2] + acc[3])
            return 0

        lax.fori_loop(0, NGRP, comb_body, 0)

    def fire_flush(co, b, fsem):
        for ch in range(CH):
            pltpu.async_copy(
                out_v.at[b, ch], out_hbm.at[ch, pl.ds(base + co, CHUNK)], fsem
            )

    def drain_flush(b, fsem):
        for ch in range(CH):
            pltpu.make_async_copy(
                out_v.at[b, ch], out_hbm.at[ch, pl.ds(base, CHUNK)], fsem
            ).wait()

    calc(0, 0)
    fire(0, gsemA)

    def pair_body(i, _):
        c0 = 2 * i
        co0 = c0 * CHUNK
        co1 = co0 + CHUNK
        calc(co1, 1)
        fire(1, gsemB)
        drain_gathers(0, gsemA)

        @pl.when(i > 0)
        def _():
            drain_flush(0, fsemA)

        combine(0)
        fire_flush(co0, 0, fsemA)

        @pl.when(i < NPAIR - 1)
        def _():
            calc(co0 + 2 * CHUNK, 0)
            fire(0, gsemA)

        drain_gathers(1, gsemB)

        @pl.when(i > 0)
        def _():
            drain_flush(1, fsemB)

        combine(1)
        fire_flush(co1, 1, fsemB)
        return 0

    lax.fori_loop(0, NPAIR, pair_body, 0)

    drain_flush(0, fsemA)
    drain_flush(1, fsemB)


def _build_table(data):
    d = data[0]
    parts = []
    for yoff, w in ((0, 1024), (1024, 512), (1536, 256), (1792, 128)):
        parts.append(
            jnp.transpose(d[:, yoff:yoff + w, :w], (1, 2, 0)).reshape(w * w, CH)
        )
    return jnp.concatenate(parts, axis=0)


def kernel(uv_inputs, data):
    table = _build_table(data)
    uvx = uv_inputs[0, 0].reshape(NPTS)
    uvy = uv_inputs[0, 1].reshape(NPTS)
    out = _sc_lookup(uvx, uvy, table)
    return out.reshape(1, CH, RES, RES)
